# baseline (speedup 1.0000x reference)
import jax
import jax.numpy as jnp
from jax import lax
from jax.experimental import pallas as pl
from jax.experimental.pallas import tpu as pltpu
from jax.experimental.pallas import tpu_sc as plsc

NUM_USER = 20000
NUM_ITEM = 30000
N_NODES = NUM_USER + NUM_ITEM
N_EDGES = 800000
DF = 128
DL = 64
DH = DL // 2

NC, NS = 2, 16
NROWS = 50176
TILE_ROWS = NROWS // NS
CHUNK = 128
E_PAD = 819200
NCHUNKS = E_PAD // CHUNK
SUP = 40
HIST_CPT = NCHUNKS // NC // NS
CONV_CPT = NCHUNKS // NS
SUPC = 16
GRP = 2
EG = GRP * CHUNK
NGS = SUPC // GRP

_MESH = plsc.VectorSubcoreMesh(
    core_axis_name="c", subcore_axis_name="s", num_cores=NC, num_subcores=NS
)


def _hist_body(rows_hbm, degp_hbm, idxbuf, ones_v, zrow, deg_sh):
    c = lax.axis_index("c")
    s = lax.axis_index("s")
    rb = s * TILE_ROWS

    @pl.loop(0, CHUNK, step=16)
    def _(i):
        ones_v[pl.ds(i, 16)] = jnp.ones((16,), jnp.float32)

    @pl.loop(0, TILE_ROWS, step=16)
    def _(i):
        zrow[pl.ds(i, 16)] = jnp.zeros((16,), jnp.float32)

    pltpu.sync_copy(zrow, deg_sh.at[pl.ds(rb, TILE_ROWS)])
    plsc.subcore_barrier()

    cb = (c * NS + s) * HIST_CPT

    @pl.loop(0, HIST_CPT, step=SUP)
    def _(u):
        pltpu.sync_copy(rows_hbm.at[pl.ds(cb + u, SUP)], idxbuf)

        @pl.loop(0, SUP)
        def _(j):
            pltpu.sync_copy(ones_v, deg_sh.at[idxbuf.at[j]], add=True)

    plsc.subcore_barrier()
    pltpu.sync_copy(deg_sh.at[pl.ds(rb, TILE_ROWS)], zrow)
    pltpu.sync_copy(zrow, degp_hbm.at[pl.ds(c * NROWS + rb, TILE_ROWS)])


_hist = pl.kernel(
    _hist_body,
    out_type=jax.ShapeDtypeStruct((NC * NROWS,), jnp.float32),
    mesh=_MESH,
    scratch_types=[
        pltpu.VMEM((SUP, CHUNK), jnp.int32),
        pltpu.VMEM((CHUNK,), jnp.float32),
        pltpu.VMEM((TILE_ROWS,), jnp.float32),
        pltpu.VMEM_SHARED((NROWS,), jnp.float32),
    ],
)


def _make_gcn_body(with_scale):
    def body(cols_hbm, rows_hbm, ytab_hbm, *rest):
        if with_scale:
            (dis2_hbm, acc_hbm, y1_hbm,
             colbuf, rowbuf, gbuf, d2b, acc_sh,
             gsem_a, gsem_b, ssem_a, ssem_b) = rest
        else:
            (acc_hbm,
             colbuf, rowbuf, gbuf, acc_sh,
             gsem_a, gsem_b, ssem_a, ssem_b) = rest
        c = lax.axis_index("c")
        s = lax.axis_index("s")
        rb = s * TILE_ROWS
        zb = gbuf.at[pl.ds(0, EG)]

        @pl.loop(0, EG)
        def _(r):
            zb[r, pl.ds(0, 16)] = jnp.zeros((16,), jnp.float32)
            zb[r, pl.ds(16, 16)] = jnp.zeros((16,), jnp.float32)

        @pl.loop(0, TILE_ROWS // EG)
        def _(k):
            pltpu.sync_copy(zb, acc_sh.at[pl.ds(rb + k * EG, EG)])

        pltpu.sync_copy(
            zb.at[pl.ds(0, TILE_ROWS % EG)],
            acc_sh.at[pl.ds(rb + (TILE_ROWS // EG) * EG, TILE_ROWS % EG)],
        )
        plsc.subcore_barrier()

        ggbase = (s * CONV_CPT) // GRP
        gsems = (gsem_a, gsem_b)
        ssems = (ssem_a, ssem_b)
        ytab_c = ytab_hbm.at[c]

        def plane(g):
            return gbuf.at[pl.ds((g % 3) * EG, EG)]

        def fire_gather(g):
            pltpu.async_copy(ytab_c.at[colbuf.at[g]], plane(g), gsems[g % 2])

        def drain_gather(parity):
            pltpu.make_async_copy(ytab_c.at[colbuf.at[0]], plane(0),
                                  gsems[parity]).wait()

        def fire_scatter(g):
            pltpu.async_copy(plane(g), acc_sh.at[rowbuf.at[g]],
                             ssems[g % 2], add=True)

        def drain_scatter(parity):
            pltpu.make_async_copy(plane(0), acc_sh.at[rowbuf.at[0]],
                                  ssems[parity]).wait()

        @pl.loop(0, CONV_CPT // SUPC)
        def _(u):
            pltpu.sync_copy(cols_hbm.at[pl.ds(ggbase + u * NGS, NGS)], colbuf)
            pltpu.sync_copy(rows_hbm.at[pl.ds(ggbase + u * NGS, NGS)], rowbuf)
            fire_gather(0)
            fire_gather(1)
            for g in range(NGS):
                drain_gather(g % 2)
                fire_scatter(g)
                if g > 0:
                    drain_scatter((g - 1) % 2)
                if g + 2 < NGS:
                    fire_gather(g + 2)
            drain_scatter((NGS - 1) % 2)

        plsc.subcore_barrier()

        acc_c = acc_hbm.at[c]
        iot = jax.lax.iota(jnp.int32, 16)

        def dump_block(k, nrows):
            pltpu.sync_copy(acc_sh.at[pl.ds(rb + k * EG, nrows)],
                            zb.at[pl.ds(0, nrows)])
            pltpu.sync_copy(zb.at[pl.ds(0, nrows)],
                            acc_c.at[pl.ds(rb + k * EG, nrows)])
            if with_scale:
                pltpu.sync_copy(dis2_hbm.at[pl.ds(rb + k * EG, nrows)],
                                d2b.at[pl.ds(0, nrows)])

                @pl.loop(0, nrows // 16)
                def _(sb):
                    dv = d2b[pl.ds(sb * 16, 16)]
                    ridx = sb * 16 + iot
                    for col in range(DH):
                        cidx = jnp.full((16,), col, jnp.int32)
                        v = plsc.load_gather(gbuf, [ridx, cidx])
                        plsc.store_scatter(gbuf, [ridx, cidx], v * dv)

                pltpu.sync_copy(zb.at[pl.ds(0, nrows)],
                                y1_hbm.at[c].at[pl.ds(rb + k * EG, nrows)])

        @pl.loop(0, TILE_ROWS // EG)
        def _(k):
            dump_block(k, EG)

        dump_block(TILE_ROWS // EG, TILE_ROWS % EG)

    return body


_GCN_OUT = jax.ShapeDtypeStruct((NC, NROWS, DH), jnp.float32)
_GCN_SCRATCH = [
    pltpu.VMEM((NGS, EG), jnp.int32),
    pltpu.VMEM((NGS, EG), jnp.int32),
    pltpu.VMEM((3 * EG, DH), jnp.float32),
]
_GCN_SEMS = [pltpu.SemaphoreType.DMA] * 4

_gcn_scale = pl.kernel(
    _make_gcn_body(True),
    out_type=[_GCN_OUT, _GCN_OUT],
    mesh=_MESH,
    scratch_types=_GCN_SCRATCH + [pltpu.VMEM((EG,), jnp.float32),
                                  pltpu.VMEM_SHARED((NROWS, DH), jnp.float32)]
    + _GCN_SEMS,
    compiler_params=pltpu.CompilerParams(use_tc_tiling_on_sc=False,
                                        needs_layout_passes=False),
)

_gcn_plain = pl.kernel(
    _make_gcn_body(False),
    out_type=_GCN_OUT,
    mesh=_MESH,
    scratch_types=_GCN_SCRATCH
    + [pltpu.VMEM_SHARED((NROWS, DH), jnp.float32)] + _GCN_SEMS,
    compiler_params=pltpu.CompilerParams(use_tc_tiling_on_sc=False,
                                        needs_layout_passes=False),
)


RM = 3000
RN = 3136


def _mlp_body(f_ref, w_ref, b_ref, w1_ref, b1_ref, o_ref):
    h = jnp.dot(f_ref[...], w_ref[...], preferred_element_type=jnp.float32)
    h = h + b_ref[...]
    h = jnp.where(h > 0, h, 0.01 * h)
    o_ref[...] = (
        jnp.dot(h, w1_ref[...], preferred_element_type=jnp.float32) + b1_ref[...]
    )


_mlp = pl.pallas_call(
    _mlp_body,
    grid=(NUM_ITEM // RM,),
    in_specs=[
        pl.BlockSpec((RM, DF), lambda i: (i, 0)),
        pl.BlockSpec((DF, 4 * DL), lambda i: (0, 0)),
        pl.BlockSpec((1, 4 * DL), lambda i: (0, 0)),
        pl.BlockSpec((4 * DL, DL), lambda i: (0, 0)),
        pl.BlockSpec((1, DL), lambda i: (0, 0)),
    ],
    out_specs=pl.BlockSpec((RM, DL), lambda i: (i, 0)),
    out_shape=jax.ShapeDtypeStruct((NUM_ITEM, DL), jnp.float32),
)


def _norm_body(xu_ref, degp_ref, xn_ref, yt_ref, dis_ref, dis2_ref):
    deg = degp_ref[:, 0:1] + degp_ref[:, 1:2]
    pos = deg > 0.0
    dis = jnp.where(pos, lax.rsqrt(deg), 0.0)
    dis2 = jnp.where(pos, 1.0 / deg, 0.0)
    x = xu_ref[...]
    nr = jnp.maximum(jnp.sqrt(jnp.sum(x * x, axis=1, keepdims=True)), 1e-12)
    xn = x / nr
    y = xn * dis
    xn_ref[...] = xn
    yt_ref[0] = y[:, :DH]
    yt_ref[1] = y[:, DH:]
    dis_ref[...] = dis
    dis2_ref[...] = dis2


_norm = pl.pallas_call(
    _norm_body,
    grid=(NROWS // RN,),
    in_specs=[
        pl.BlockSpec((RN, DL), lambda i: (i, 0)),
        pl.BlockSpec((RN, 2), lambda i: (i, 0)),
    ],
    out_specs=[
        pl.BlockSpec((RN, DL), lambda i: (i, 0)),
        pl.BlockSpec((2, RN, DH), lambda i: (0, i, 0)),
        pl.BlockSpec((RN, 1), lambda i: (i, 0)),
        pl.BlockSpec((RN, 1), lambda i: (i, 0)),
    ],
    out_shape=[
        jax.ShapeDtypeStruct((NROWS, DL), jnp.float32),
        jax.ShapeDtypeStruct((2, NROWS, DH), jnp.float32),
        jax.ShapeDtypeStruct((NROWS, 1), jnp.float32),
        jax.ShapeDtypeStruct((NROWS, 1), jnp.float32),
    ],
)


def _scale_body(a_ref, dis2_ref, o_ref):
    o_ref[0] = a_ref[0] * dis2_ref[...]
    o_ref[1] = a_ref[1] * dis2_ref[...]


_scale = pl.pallas_call(
    _scale_body,
    grid=(NROWS // RN,),
    in_specs=[
        pl.BlockSpec((2, RN, DH), lambda i: (0, i, 0)),
        pl.BlockSpec((RN, 1), lambda i: (i, 0)),
    ],
    out_specs=pl.BlockSpec((2, RN, DH), lambda i: (0, i, 0)),
    out_shape=jax.ShapeDtypeStruct((2, NROWS, DH), jnp.float32),
)


def _comb_body(xn_ref, a1_ref, a2_ref, dis_ref, o_ref):
    h0 = a1_ref[0] + a2_ref[0]
    h1 = a1_ref[1] + a2_ref[1]
    hh = jnp.concatenate([h0, h1], axis=1)
    o_ref[...] = xn_ref[...] + dis_ref[...] * hh


_comb = pl.pallas_call(
    _comb_body,
    grid=(NROWS // RN,),
    in_specs=[
        pl.BlockSpec((RN, DL), lambda i: (i, 0)),
        pl.BlockSpec((2, RN, DH), lambda i: (0, i, 0)),
        pl.BlockSpec((2, RN, DH), lambda i: (0, i, 0)),
        pl.BlockSpec((RN, 1), lambda i: (i, 0)),
    ],
    out_specs=pl.BlockSpec((RN, DL), lambda i: (i, 0)),
    out_shape=jax.ShapeDtypeStruct((NROWS, DL), jnp.float32),
)


def kernel(edge_index_drop, edge_index, features, preference,
           W_mlp, b_mlp, W_mlp1, b_mlp1):
    rows = edge_index[0].astype(jnp.int32)
    cols = edge_index[1].astype(jnp.int32)
    pad = E_PAD - N_EDGES
    rows_p = jnp.concatenate(
        [rows, jnp.full((pad,), N_NODES, jnp.int32)]).reshape(NCHUNKS, CHUNK)
    cols_p = jnp.concatenate(
        [cols, jnp.zeros((pad,), jnp.int32)]).reshape(NCHUNKS, CHUNK)

    cols3 = cols_p.reshape(NCHUNKS // GRP, EG)
    rows3 = rows_p.reshape(NCHUNKS // GRP, EG)

    degp = _hist(rows_p)
    temp = _mlp(features, W_mlp, b_mlp.reshape(1, -1), W_mlp1,
                b_mlp1.reshape(1, -1))
    xu = jnp.concatenate(
        [preference, temp, jnp.zeros((NROWS - N_NODES, DL), jnp.float32)],
        axis=0)
    degp_t = jnp.transpose(degp.reshape(NC, NROWS))
    xn, yt, dis, dis2 = _norm(xu, degp_t)
    acc1, y1 = _gcn_scale(cols3, rows3, yt, dis2.reshape(NROWS))
    acc2 = _gcn_plain(cols3, rows3, y1)
    out = _comb(xn, acc1, acc2, dis)
    return out[:N_NODES], preference


# speedup vs baseline: 1.0533x; 1.0533x over previous
import jax
import jax.numpy as jnp
from jax import lax
from jax.experimental import pallas as pl
from jax.experimental.pallas import tpu as pltpu
from jax.experimental.pallas import tpu_sc as plsc

NUM_USER = 20000
NUM_ITEM = 30000
N_NODES = NUM_USER + NUM_ITEM
N_EDGES = 800000
DF = 128
DL = 64
DH = DL // 2

NC, NS = 2, 16
NROWS = 50176
TILE_ROWS = NROWS // NS
CHUNK = 128
E_PAD = 819200
NCHUNKS = E_PAD // CHUNK
SUP = 40
HIST_CPT = NCHUNKS // NC // NS
CONV_CPT = NCHUNKS // NS
SUPC = 16
GRP = 2
EG = GRP * CHUNK
NGS = SUPC // GRP

_MESH = plsc.VectorSubcoreMesh(
    core_axis_name="c", subcore_axis_name="s", num_cores=NC, num_subcores=NS
)


def _hist_body(rows_hbm, degp_hbm, idxbuf, ones_v, zrow, deg_sh):
    c = lax.axis_index("c")
    s = lax.axis_index("s")
    rb = s * TILE_ROWS

    @pl.loop(0, CHUNK, step=16)
    def _(i):
        ones_v[pl.ds(i, 16)] = jnp.ones((16,), jnp.float32)

    @pl.loop(0, TILE_ROWS, step=16)
    def _(i):
        zrow[pl.ds(i, 16)] = jnp.zeros((16,), jnp.float32)

    pltpu.sync_copy(zrow, deg_sh.at[pl.ds(rb, TILE_ROWS)])
    plsc.subcore_barrier()

    cb = (c * NS + s) * HIST_CPT

    @pl.loop(0, HIST_CPT, step=SUP)
    def _(u):
        pltpu.sync_copy(rows_hbm.at[pl.ds(cb + u, SUP)], idxbuf)

        @pl.loop(0, SUP)
        def _(j):
            pltpu.sync_copy(ones_v, deg_sh.at[idxbuf.at[j]], add=True)

    plsc.subcore_barrier()
    pltpu.sync_copy(deg_sh.at[pl.ds(rb, TILE_ROWS)], zrow)
    pltpu.sync_copy(zrow, degp_hbm.at[pl.ds(c * NROWS + rb, TILE_ROWS)])


_hist = pl.kernel(
    _hist_body,
    out_type=jax.ShapeDtypeStruct((NC * NROWS,), jnp.float32),
    mesh=_MESH,
    scratch_types=[
        pltpu.VMEM((SUP, CHUNK), jnp.int32),
        pltpu.VMEM((CHUNK,), jnp.float32),
        pltpu.VMEM((TILE_ROWS,), jnp.float32),
        pltpu.VMEM_SHARED((NROWS,), jnp.float32),
    ],
)


def _make_gcn_body(with_scale):
    def body(cols_hbm, rows_hbm, ytab_hbm, *rest):
        if with_scale:
            (dis2_hbm, acc_hbm, y1_hbm,
             colbuf, rowbuf, gbuf, d2b, acc_sh,
             gsem_a, gsem_b, ssem_a, ssem_b) = rest
        else:
            (acc_hbm,
             colbuf, rowbuf, gbuf, acc_sh,
             gsem_a, gsem_b, ssem_a, ssem_b) = rest
        c = lax.axis_index("c")
        s = lax.axis_index("s")
        rb = s * TILE_ROWS
        zb = gbuf.at[pl.ds(0, EG)]

        @pl.loop(0, EG)
        def _(r):
            zb[r, pl.ds(0, 16)] = jnp.zeros((16,), jnp.float32)
            zb[r, pl.ds(16, 16)] = jnp.zeros((16,), jnp.float32)

        @pl.loop(0, TILE_ROWS // EG)
        def _(k):
            pltpu.sync_copy(zb, acc_sh.at[pl.ds(rb + k * EG, EG)])

        pltpu.sync_copy(
            zb.at[pl.ds(0, TILE_ROWS % EG)],
            acc_sh.at[pl.ds(rb + (TILE_ROWS // EG) * EG, TILE_ROWS % EG)],
        )
        plsc.subcore_barrier()

        ggbase = (s * CONV_CPT) // GRP
        gsems = (gsem_a, gsem_b)
        ssems = (ssem_a, ssem_b)
        ytab_c = ytab_hbm.at[c]

        def plane(g):
            return gbuf.at[pl.ds((g % 3) * EG, EG)]

        def fire_gather(g):
            pltpu.async_copy(ytab_c.at[colbuf.at[g]], plane(g), gsems[g % 2])

        def drain_gather(parity):
            pltpu.make_async_copy(ytab_c.at[colbuf.at[0]], plane(0),
                                  gsems[parity]).wait()

        def fire_scatter(g):
            pltpu.async_copy(plane(g), acc_sh.at[rowbuf.at[g]],
                             ssems[g % 2], add=True)

        def drain_scatter(parity):
            pltpu.make_async_copy(plane(0), acc_sh.at[rowbuf.at[0]],
                                  ssems[parity]).wait()

        @pl.loop(0, CONV_CPT // SUPC)
        def _(u):
            pltpu.sync_copy(cols_hbm.at[pl.ds(ggbase + u * NGS, NGS)], colbuf)
            pltpu.sync_copy(rows_hbm.at[pl.ds(ggbase + u * NGS, NGS)], rowbuf)
            fire_gather(0)
            fire_gather(1)
            for g in range(NGS):
                drain_gather(g % 2)
                fire_scatter(g)
                if g > 0:
                    drain_scatter((g - 1) % 2)
                if g + 2 < NGS:
                    fire_gather(g + 2)
            drain_scatter((NGS - 1) % 2)

        plsc.subcore_barrier()

        acc_c = acc_hbm.at[c]
        iot = jax.lax.iota(jnp.int32, 16)

        def dump_block(k, nrows):
            pltpu.sync_copy(acc_sh.at[pl.ds(rb + k * EG, nrows)],
                            zb.at[pl.ds(0, nrows)])
            pltpu.sync_copy(zb.at[pl.ds(0, nrows)],
                            acc_c.at[pl.ds(rb + k * EG, nrows)])
            if with_scale:
                pltpu.sync_copy(dis2_hbm.at[pl.ds(rb + k * EG, nrows)],
                                d2b.at[pl.ds(0, nrows)])

                @pl.loop(0, nrows // 16)
                def _(sb):
                    dv = d2b[pl.ds(sb * 16, 16)]
                    ridx = sb * 16 + iot
                    for col in range(DH):
                        cidx = jnp.full((16,), col, jnp.int32)
                        v = plsc.load_gather(gbuf, [ridx, cidx])
                        plsc.store_scatter(gbuf, [ridx, cidx], v * dv)

                pltpu.sync_copy(zb.at[pl.ds(0, nrows)],
                                y1_hbm.at[c].at[pl.ds(rb + k * EG, nrows)])

        @pl.loop(0, TILE_ROWS // EG)
        def _(k):
            dump_block(k, EG)

        dump_block(TILE_ROWS // EG, TILE_ROWS % EG)

    return body


_GCN_OUT = jax.ShapeDtypeStruct((NC, NROWS, DH), jnp.float32)
_GCN_SCRATCH = [
    pltpu.VMEM((NGS, EG), jnp.int32),
    pltpu.VMEM((NGS, EG), jnp.int32),
    pltpu.VMEM((3 * EG, DH), jnp.float32),
]
_GCN_SEMS = [pltpu.SemaphoreType.DMA] * 4

_gcn_plain = pl.kernel(
    _make_gcn_body(False),
    out_type=_GCN_OUT,
    mesh=_MESH,
    scratch_types=_GCN_SCRATCH
    + [pltpu.VMEM_SHARED((NROWS, DH), jnp.float32)] + _GCN_SEMS,
    compiler_params=pltpu.CompilerParams(use_tc_tiling_on_sc=False),
)


RM = 3000
RN = 3136


def _mlp_body(f_ref, w_ref, b_ref, w1_ref, b1_ref, o_ref):
    h = jnp.dot(f_ref[...], w_ref[...], preferred_element_type=jnp.float32)
    h = h + b_ref[...]
    h = jnp.where(h > 0, h, 0.01 * h)
    o_ref[...] = (
        jnp.dot(h, w1_ref[...], preferred_element_type=jnp.float32) + b1_ref[...]
    )


_mlp = pl.pallas_call(
    _mlp_body,
    grid=(NUM_ITEM // RM,),
    in_specs=[
        pl.BlockSpec((RM, DF), lambda i: (i, 0)),
        pl.BlockSpec((DF, 4 * DL), lambda i: (0, 0)),
        pl.BlockSpec((1, 4 * DL), lambda i: (0, 0)),
        pl.BlockSpec((4 * DL, DL), lambda i: (0, 0)),
        pl.BlockSpec((1, DL), lambda i: (0, 0)),
    ],
    out_specs=pl.BlockSpec((RM, DL), lambda i: (i, 0)),
    out_shape=jax.ShapeDtypeStruct((NUM_ITEM, DL), jnp.float32),
)


def _norm_body(xu_ref, degp_ref, xn_ref, yt_ref, dis_ref, dis2_ref):
    deg = degp_ref[:, 0:1] + degp_ref[:, 1:2]
    pos = deg > 0.0
    dis = jnp.where(pos, lax.rsqrt(deg), 0.0)
    dis2 = jnp.where(pos, 1.0 / deg, 0.0)
    x = xu_ref[...]
    nr = jnp.maximum(jnp.sqrt(jnp.sum(x * x, axis=1, keepdims=True)), 1e-12)
    xn = x / nr
    y = xn * dis
    xn_ref[...] = xn
    yt_ref[0] = y[:, :DH]
    yt_ref[1] = y[:, DH:]
    dis_ref[...] = dis
    dis2_ref[...] = dis2


_norm = pl.pallas_call(
    _norm_body,
    grid=(NROWS // RN,),
    in_specs=[
        pl.BlockSpec((RN, DL), lambda i: (i, 0)),
        pl.BlockSpec((RN, 2), lambda i: (i, 0)),
    ],
    out_specs=[
        pl.BlockSpec((RN, DL), lambda i: (i, 0)),
        pl.BlockSpec((2, RN, DH), lambda i: (0, i, 0)),
        pl.BlockSpec((RN, 1), lambda i: (i, 0)),
        pl.BlockSpec((RN, 1), lambda i: (i, 0)),
    ],
    out_shape=[
        jax.ShapeDtypeStruct((NROWS, DL), jnp.float32),
        jax.ShapeDtypeStruct((2, NROWS, DH), jnp.float32),
        jax.ShapeDtypeStruct((NROWS, 1), jnp.float32),
        jax.ShapeDtypeStruct((NROWS, 1), jnp.float32),
    ],
)


def _scale_body(a_ref, dis2_ref, o_ref):
    o_ref[0] = a_ref[0] * dis2_ref[...]
    o_ref[1] = a_ref[1] * dis2_ref[...]


_scale = pl.pallas_call(
    _scale_body,
    grid=(NROWS // RN,),
    in_specs=[
        pl.BlockSpec((2, RN, DH), lambda i: (0, i, 0)),
        pl.BlockSpec((RN, 1), lambda i: (i, 0)),
    ],
    out_specs=pl.BlockSpec((2, RN, DH), lambda i: (0, i, 0)),
    out_shape=jax.ShapeDtypeStruct((2, NROWS, DH), jnp.float32),
)


def _comb_body(xn_ref, a1_ref, a2_ref, dis_ref, o_ref):
    h0 = a1_ref[0] + a2_ref[0]
    h1 = a1_ref[1] + a2_ref[1]
    hh = jnp.concatenate([h0, h1], axis=1)
    o_ref[...] = xn_ref[...] + dis_ref[...] * hh


_comb = pl.pallas_call(
    _comb_body,
    grid=(NROWS // RN,),
    in_specs=[
        pl.BlockSpec((RN, DL), lambda i: (i, 0)),
        pl.BlockSpec((2, RN, DH), lambda i: (0, i, 0)),
        pl.BlockSpec((2, RN, DH), lambda i: (0, i, 0)),
        pl.BlockSpec((RN, 1), lambda i: (i, 0)),
    ],
    out_specs=pl.BlockSpec((RN, DL), lambda i: (i, 0)),
    out_shape=jax.ShapeDtypeStruct((NROWS, DL), jnp.float32),
)


def kernel(edge_index_drop, edge_index, features, preference,
           W_mlp, b_mlp, W_mlp1, b_mlp1):
    rows = edge_index[0].astype(jnp.int32)
    cols = edge_index[1].astype(jnp.int32)
    pad = E_PAD - N_EDGES
    rows_p = jnp.concatenate(
        [rows, jnp.full((pad,), N_NODES, jnp.int32)]).reshape(NCHUNKS, CHUNK)
    cols_p = jnp.concatenate(
        [cols, jnp.zeros((pad,), jnp.int32)]).reshape(NCHUNKS, CHUNK)

    cols3 = cols_p.reshape(NCHUNKS // GRP, EG)
    rows3 = rows_p.reshape(NCHUNKS // GRP, EG)

    degp = _hist(rows_p)
    temp = _mlp(features, W_mlp, b_mlp.reshape(1, -1), W_mlp1,
                b_mlp1.reshape(1, -1))
    xu = jnp.concatenate(
        [preference, temp, jnp.zeros((NROWS - N_NODES, DL), jnp.float32)],
        axis=0)
    degp_t = jnp.transpose(degp.reshape(NC, NROWS))
    xn, yt, dis, dis2 = _norm(xu, degp_t)
    acc1 = _gcn_plain(cols3, rows3, yt)
    y1 = _scale(acc1, dis2)
    acc2 = _gcn_plain(cols3, rows3, y1)
    out = _comb(xn, acc1, acc2, dis)
    return out[:N_NODES], preference
